# Initial kernel scaffold; baseline (speedup 1.0000x reference)
import jax
import jax.numpy as jnp
from jax.experimental import pallas as pl


def kernel(features, edge_index, W, b):
    raise NotImplementedError("write your pallas kernel here")


# speedup vs baseline: 16.9631x; 16.9631x over previous
import functools

import jax
import jax.numpy as jnp
from jax import lax
from jax.experimental import pallas as pl
from jax.experimental.pallas import tpu as pltpu
from jax.experimental.pallas import tpu_sc as plsc

N_NODES = 10000
N_EDGES = 320000
FEAT = 128

NC, NS = 2, 16
NW = NC * NS
EPW = N_EDGES // NW
K = 80
NCHUNK = EPW // K
NP = 10240
RPT = NP // NS
ZR = 128
DEGW = 16

_mesh = plsc.VectorSubcoreMesh(core_axis_name="c", subcore_axis_name="s")


@functools.partial(
    pl.kernel,
    out_type=jax.ShapeDtypeStruct((NC, NP, DEGW), jnp.float32),
    mesh=_mesh,
    scratch_types=[
        pltpu.VMEM_SHARED((NP, DEGW), jnp.float32),
        pltpu.VMEM((K,), jnp.int32),
        pltpu.VMEM((K, DEGW), jnp.float32),
        pltpu.VMEM((ZR, DEGW), jnp.float32),
    ],
)
def _deg_kernel(col_hbm, deg_hbm, deg_sh, col_buf, ones_buf, zbuf):
    cid = lax.axis_index("c")
    sid = lax.axis_index("s")
    wid = sid * NC + cid

    def fill_ones(i, _):
        ones_buf[i] = jnp.ones((DEGW,), jnp.float32)
        return 0

    lax.fori_loop(0, K, fill_ones, 0)

    def fill_zero(i, _):
        zbuf[i] = jnp.zeros((DEGW,), jnp.float32)
        return 0

    lax.fori_loop(0, ZR, fill_zero, 0)

    for j in range(RPT // ZR):
        pltpu.sync_copy(zbuf, deg_sh.at[pl.ds(sid * RPT + j * ZR, ZR)])
    plsc.subcore_barrier()

    def body(j, _):
        off = wid * EPW + j * K
        pltpu.sync_copy(col_hbm.at[pl.ds(off, K)], col_buf)
        pltpu.sync_copy(ones_buf, deg_sh.at[col_buf], add=True)
        return 0

    lax.fori_loop(0, NCHUNK, body, 0)
    plsc.subcore_barrier()

    pltpu.sync_copy(
        deg_sh.at[pl.ds(sid * RPT, RPT)],
        deg_hbm.at[cid, pl.ds(sid * RPT, RPT)],
    )


@functools.partial(
    pl.kernel,
    out_type=jax.ShapeDtypeStruct((NC, NP, FEAT), jnp.float32),
    mesh=_mesh,
    scratch_types=[
        pltpu.VMEM_SHARED((NP, FEAT), jnp.float32),
        pltpu.VMEM((K,), jnp.int32),
        pltpu.VMEM((K,), jnp.int32),
        pltpu.VMEM((K, FEAT), jnp.float32),
        pltpu.VMEM((ZR, FEAT), jnp.float32),
        pltpu.SemaphoreType.DMA,
    ],
)
def _scatter_kernel(y_hbm, row_hbm, col_hbm, acc_hbm,
                    acc_sh, row_buf, col_buf, gbuf, zbuf, sem):
    cid = lax.axis_index("c")
    sid = lax.axis_index("s")
    wid = sid * NC + cid

    def fill_zero(i, _):
        for t in range(FEAT // 16):
            zbuf[i, pl.ds(t * 16, 16)] = jnp.zeros((16,), jnp.float32)
        return 0

    lax.fori_loop(0, ZR, fill_zero, 0)

    for j in range(RPT // ZR):
        pltpu.sync_copy(zbuf, acc_sh.at[pl.ds(sid * RPT + j * ZR, ZR)])
    plsc.subcore_barrier()

    def body(j, _):
        off = wid * EPW + j * K
        pltpu.sync_copy(row_hbm.at[pl.ds(off, K)], row_buf)
        pltpu.sync_copy(col_hbm.at[pl.ds(off, K)], col_buf)
        pltpu.async_copy(y_hbm.at[row_buf], gbuf, sem).wait()
        pltpu.sync_copy(gbuf, acc_sh.at[col_buf], add=True)
        return 0

    lax.fori_loop(0, NCHUNK, body, 0)
    plsc.subcore_barrier()

    pltpu.sync_copy(
        acc_sh.at[pl.ds(sid * RPT, RPT)],
        acc_hbm.at[cid, pl.ds(sid * RPT, RPT)],
    )


BN = 400


def _transform_body(x_ref, w_ref, deg_ref, y_ref):
    deg = deg_ref[0, :, 0] + deg_ref[1, :, 0] + 1.0
    d = lax.rsqrt(deg)
    y_ref[...] = jnp.dot(
        x_ref[...], w_ref[...], preferred_element_type=jnp.float32
    ) * d[:, None]


def _final_body(acc_ref, y_ref, deg_ref, b_ref, out_ref):
    deg = deg_ref[0, :, 0] + deg_ref[1, :, 0] + 1.0
    d = lax.rsqrt(deg)
    out_ref[...] = (
        d[:, None] * (acc_ref[0] + acc_ref[1] + y_ref[...]) + b_ref[...]
    )


def kernel(features, edge_index, W, b):
    row = edge_index[0].astype(jnp.int32)
    col = edge_index[1].astype(jnp.int32)

    deg = _deg_kernel(col)

    y = pl.pallas_call(
        _transform_body,
        grid=(N_NODES // BN,),
        in_specs=[
            pl.BlockSpec((BN, FEAT), lambda i: (i, 0)),
            pl.BlockSpec((FEAT, FEAT), lambda i: (0, 0)),
            pl.BlockSpec((NC, BN, DEGW), lambda i: (0, i, 0)),
        ],
        out_specs=pl.BlockSpec((BN, FEAT), lambda i: (i, 0)),
        out_shape=jax.ShapeDtypeStruct((N_NODES, FEAT), jnp.float32),
    )(features, W, deg)

    acc = _scatter_kernel(y, row, col)

    out = pl.pallas_call(
        _final_body,
        grid=(N_NODES // BN,),
        in_specs=[
            pl.BlockSpec((NC, BN, FEAT), lambda i: (0, i, 0)),
            pl.BlockSpec((BN, FEAT), lambda i: (i, 0)),
            pl.BlockSpec((NC, BN, DEGW), lambda i: (0, i, 0)),
            pl.BlockSpec((1, FEAT), lambda i: (0, 0)),
        ],
        out_specs=pl.BlockSpec((BN, FEAT), lambda i: (i, 0)),
        out_shape=jax.ShapeDtypeStruct((N_NODES, FEAT), jnp.float32),
    )(acc, y, deg, jnp.reshape(b, (1, FEAT)))
    return out


# speedup vs baseline: 40.6340x; 2.3954x over previous
import functools

import jax
import jax.numpy as jnp
from jax import lax
from jax.experimental import pallas as pl
from jax.experimental.pallas import tpu as pltpu
from jax.experimental.pallas import tpu_sc as plsc

N_NODES = 10000
N_EDGES = 320000
FEAT = 128

NC, NS = 2, 16
NW = NC * NS
EPW = N_EDGES // NW
NP = 10240
RPT = NP // NS
DEGW = 16

K = 80
NCH = EPW // K
DEPTH = 4
NCH_MAIN = (NCH // DEPTH) * DEPTH

_mesh = plsc.VectorSubcoreMesh(core_axis_name="c", subcore_axis_name="s")


@functools.partial(
    pl.kernel,
    out_type=jax.ShapeDtypeStruct((NC, NP, DEGW), jnp.float32),
    mesh=_mesh,
    compiler_params=pltpu.CompilerParams(needs_layout_passes=False),
    scratch_types=[
        pltpu.VMEM_SHARED((NS, NP), jnp.float32),
        pltpu.VMEM((EPW,), jnp.int32),
        pltpu.VMEM((NP,), jnp.float32),
        pltpu.VMEM((NS, RPT), jnp.float32),
        pltpu.VMEM((RPT, DEGW), jnp.float32),
    ],
)
def _deg_kernel(col_hbm, deg_hbm, stage_sh, col_buf, hist, pulled, summed):
    cid = lax.axis_index("c")
    sid = lax.axis_index("s")
    wid = sid * NC + cid

    def zero_hist(i, _):
        hist[pl.ds(i * 16, 16)] = jnp.zeros((16,), jnp.float32)
        return 0

    lax.fori_loop(0, NP // 16, zero_hist, 0)
    pltpu.sync_copy(col_hbm.at[pl.ds(wid * EPW, EPW)], col_buf)

    def body(i, _):
        idx = col_buf[pl.ds(i * 16, 16)]
        cnt, last = plsc.scan_count(idx)
        cur = plsc.load_gather(hist, [idx], mask=last)
        plsc.store_scatter(hist, [idx], cur + cnt.astype(jnp.float32),
                           mask=last)
        return 0

    lax.fori_loop(0, EPW // 16, body, 0)
    pltpu.sync_copy(hist, stage_sh.at[sid])
    plsc.subcore_barrier()
    pltpu.sync_copy(stage_sh.at[:, pl.ds(sid * RPT, RPT)], pulled)

    def accum(i, _):
        s = jnp.zeros((16,), jnp.float32)
        for t in range(NS):
            s = s + pulled[t, pl.ds(i * 16, 16)]
        for j in range(16):
            summed[i * 16 + j] = jnp.broadcast_to(s[j], (DEGW,))
        return 0

    lax.fori_loop(0, RPT // 16, accum, 0)
    pltpu.sync_copy(summed, deg_hbm.at[cid, pl.ds(sid * RPT, RPT)])


@functools.partial(
    pl.kernel,
    out_type=jax.ShapeDtypeStruct((NC, NP, FEAT), jnp.float32),
    mesh=_mesh,
    scratch_types=[
        pltpu.VMEM_SHARED((NP, FEAT), jnp.float32),
        pltpu.VMEM((DEPTH, K), jnp.int32),
        pltpu.VMEM((DEPTH, K), jnp.int32),
        [pltpu.VMEM((K, FEAT), jnp.float32)] * DEPTH,
        [pltpu.SemaphoreType.DMA] * DEPTH,
        [pltpu.SemaphoreType.DMA] * DEPTH,
        [pltpu.SemaphoreType.DMA] * DEPTH,
        [pltpu.SemaphoreType.DMA] * DEPTH,
    ],
)
def _scatter_kernel(y_hbm, row_hbm, col_hbm, acc_hbm,
                    acc_sh, row_ring, col_ring, gbufs,
                    irsems, icsems, gsems, ssems):
    cid = lax.axis_index("c")
    sid = lax.axis_index("s")
    wid = sid * NC + cid
    base = wid * EPW

    def fill_zero(i, _):
        for t in range(FEAT // 16):
            gbufs[0][i, pl.ds(t * 16, 16)] = jnp.zeros((16,), jnp.float32)
        return 0

    lax.fori_loop(0, K, fill_zero, 0)
    for j in range(RPT // K):
        pltpu.sync_copy(gbufs[0], acc_sh.at[pl.ds(sid * RPT + j * K, K)])
    plsc.subcore_barrier()

    def idx_start(c, s):
        pltpu.async_copy(row_hbm.at[pl.ds(base + c * K, K)],
                         row_ring.at[s], irsems[s])
        pltpu.async_copy(col_hbm.at[pl.ds(base + c * K, K)],
                         col_ring.at[s], icsems[s])

    def idx_wait(s):
        pltpu.make_async_copy(row_hbm.at[pl.ds(base, K)],
                              row_ring.at[s], irsems[s]).wait()
        pltpu.make_async_copy(col_hbm.at[pl.ds(base, K)],
                              col_ring.at[s], icsems[s]).wait()

    def gather_start(s):
        pltpu.async_copy(y_hbm.at[row_ring.at[s]], gbufs[s], gsems[s])

    def gather_wait(s):
        pltpu.make_async_copy(y_hbm.at[row_ring.at[s]],
                              gbufs[s], gsems[s]).wait()

    def scatter_start(s):
        pltpu.async_copy(gbufs[s], acc_sh.at[col_ring.at[s]], ssems[s],
                         add=True)

    def scatter_wait(s):
        pltpu.make_async_copy(gbufs[s], acc_sh.at[col_ring.at[s]],
                              ssems[s]).wait()

    idx_start(0, 0)
    idx_start(1, 1)
    idx_wait(0)
    gather_start(0)

    def body(i, _):
        for off in range(DEPTH):
            c = i * DEPTH + off

            @pl.when(c >= 2)
            def _():
                scatter_wait((off + 2) % DEPTH)

            @pl.when(c + 2 < NCH)
            def _():
                idx_start(c + 2, (off + 2) % DEPTH)

            @pl.when(c + 1 < NCH)
            def _():
                idx_wait((off + 1) % DEPTH)
                gather_start((off + 1) % DEPTH)

            gather_wait(off)
            scatter_start(off)
        return 0

    lax.fori_loop(0, NCH_MAIN // DEPTH, body, 0)
    last = (NCH - 1) % DEPTH
    scatter_wait((last + 2) % DEPTH)
    gather_wait(last)
    scatter_start(last)
    scatter_wait((NCH - 2) % DEPTH)
    scatter_wait(last)
    plsc.subcore_barrier()

    pltpu.sync_copy(
        acc_sh.at[pl.ds(sid * RPT, RPT)],
        acc_hbm.at[cid, pl.ds(sid * RPT, RPT)],
    )


BN = 400


def _matmul_body(x_ref, w_ref, xw_ref):
    xw_ref[...] = jnp.dot(
        x_ref[...], w_ref[...], preferred_element_type=jnp.float32
    )


def _scale_body(xw_ref, deg_ref, y_ref):
    deg = deg_ref[0, :, 0] + deg_ref[1, :, 0] + 1.0
    d = lax.rsqrt(deg)
    y_ref[...] = xw_ref[...] * d[:, None]


def _final_body(acc_ref, y_ref, deg_ref, b_ref, out_ref):
    deg = deg_ref[0, :, 0] + deg_ref[1, :, 0] + 1.0
    d = lax.rsqrt(deg)
    out_ref[...] = (
        d[:, None] * (acc_ref[0] + acc_ref[1] + y_ref[...]) + b_ref[...]
    )


def kernel(features, edge_index, W, b):
    row = edge_index[0].astype(jnp.int32)
    col = edge_index[1].astype(jnp.int32)
    deg = _deg_kernel(col)

    xw = pl.pallas_call(
        _matmul_body,
        grid=(N_NODES // BN,),
        in_specs=[
            pl.BlockSpec((BN, FEAT), lambda i: (i, 0)),
            pl.BlockSpec((FEAT, FEAT), lambda i: (0, 0)),
        ],
        out_specs=pl.BlockSpec((BN, FEAT), lambda i: (i, 0)),
        out_shape=jax.ShapeDtypeStruct((N_NODES, FEAT), jnp.float32),
    )(features, W)

    y = pl.pallas_call(
        _scale_body,
        grid=(N_NODES // BN,),
        in_specs=[
            pl.BlockSpec((BN, FEAT), lambda i: (i, 0)),
            pl.BlockSpec((NC, BN, DEGW), lambda i: (0, i, 0)),
        ],
        out_specs=pl.BlockSpec((BN, FEAT), lambda i: (i, 0)),
        out_shape=jax.ShapeDtypeStruct((N_NODES, FEAT), jnp.float32),
    )(xw, deg)

    acc = _scatter_kernel(y, row, col)

    out = pl.pallas_call(
        _final_body,
        grid=(N_NODES // BN,),
        in_specs=[
            pl.BlockSpec((NC, BN, FEAT), lambda i: (0, i, 0)),
            pl.BlockSpec((BN, FEAT), lambda i: (i, 0)),
            pl.BlockSpec((NC, BN, DEGW), lambda i: (0, i, 0)),
            pl.BlockSpec((1, FEAT), lambda i: (0, 0)),
        ],
        out_specs=pl.BlockSpec((BN, FEAT), lambda i: (i, 0)),
        out_shape=jax.ShapeDtypeStruct((N_NODES, FEAT), jnp.float32),
    )(acc, y, deg, jnp.reshape(b, (1, FEAT)))
    return out
